# baseline (speedup 1.0000x reference)
import functools

import jax
import jax.numpy as jnp
from jax import lax
from jax.experimental import pallas as pl
from jax.experimental.pallas import tpu as pltpu
from jax.experimental.pallas import tpu_sc as plsc

_NC, _NS, _L = 2, 16, 16
_NW = _NC * _NS


def _make_sc_call(M, C):
    R = M // _NW
    n_groups = R // _L
    mesh = plsc.VectorSubcoreMesh(core_axis_name="c", subcore_axis_name="s")

    @functools.partial(
        pl.kernel,
        out_type=[
            jax.ShapeDtypeStruct((M, C), jnp.float32),
            jax.ShapeDtypeStruct((M, C), jnp.float32),
        ],
        mesh=mesh,
        scratch_types=[
            pltpu.VMEM((R,), jnp.float32),
            pltpu.VMEM((R,), jnp.float32),
            pltpu.VMEM((_L,), jnp.float32),
            pltpu.SemaphoreType.DMA,
            pltpu.SemaphoreType.DMA,
        ],
    )
    def sc_call(thr_hbm, m0_hbm, m1_hbm, x0_hbm, x1_hbm, o0_hbm, o1_hbm,
                m0all, m1all, thr_v, sem0, sem1):
        wid = lax.axis_index("s") * _NC + lax.axis_index("c")
        base_row = wid * R
        pltpu.sync_copy(thr_hbm, thr_v)
        pltpu.sync_copy(m0_hbm.at[pl.ds(base_row, R)], m0all)
        pltpu.sync_copy(m1_hbm.at[pl.ds(base_row, R)], m1all)
        thr_s = thr_v[...][0]

        def group_body(g, carry):
            mv0 = m0all[pl.ds(g * _L, _L)]
            mv1 = m1all[pl.ds(g * _L, _L)]
            row0 = base_row + g * _L
            for t in range(_L):
                row = row0 + t
                dst0 = o0_hbm.at[pl.ds(row, 1)]
                dst1 = o1_hbm.at[pl.ds(row, 1)]
                s0 = x0_hbm.at[pl.ds(row, 1)]
                s1 = x1_hbm.at[pl.ds(row, 1)]
                k0 = mv0[t] >= thr_s
                k1 = mv1[t] >= thr_s

                @pl.when(k0)
                def _(s0=s0, dst0=dst0):
                    pltpu.make_async_copy(s0, dst0, sem0).start()

                @pl.when(jnp.logical_not(k0))
                def _(s1=s1, dst0=dst0):
                    pltpu.make_async_copy(s1, dst0, sem0).start()

                @pl.when(k1)
                def _(s1=s1, dst1=dst1):
                    pltpu.make_async_copy(s1, dst1, sem1).start()

                @pl.when(jnp.logical_not(k1))
                def _(s0=s0, dst1=dst1):
                    pltpu.make_async_copy(s0, dst1, sem1).start()
            return carry

        lax.fori_loop(0, n_groups, group_body, 0)

        def wait_body(r, carry):
            pltpu.make_async_copy(
                x0_hbm.at[pl.ds(0, 1)], o0_hbm.at[pl.ds(0, 1)], sem0).wait()
            pltpu.make_async_copy(
                x0_hbm.at[pl.ds(0, 1)], o1_hbm.at[pl.ds(0, 1)], sem1).wait()
            return carry

        lax.fori_loop(0, R, wait_body, 0)

    return sc_call


def kernel(x0, x1, mask0, mask1, mask_threshold):
    B, N, C = x0.shape
    M = B * N
    x0f = x0.reshape(M, C)
    x1f = x1.reshape(M, C)
    m0 = mask0.reshape(M)
    m1 = mask1.reshape(M)
    thr = jnp.full((_L,), mask_threshold, jnp.float32)
    o0, o1 = _make_sc_call(M, C)(thr, m0, m1, x0f, x1f)
    return o0.reshape(B, N, C), o1.reshape(B, N, C)


# speedup vs baseline: 37.9199x; 37.9199x over previous
import functools

import jax
import jax.numpy as jnp
from jax import lax
from jax.experimental import pallas as pl
from jax.experimental.pallas import tpu as pltpu
from jax.experimental.pallas import tpu_sc as plsc

_NC, _NS, _L = 2, 16, 16
_NW = _NC * _NS
_T = 16
_H = _T // 2


def _make_sc_call(M, C):
    R = M // _NW
    n_chunks = R // _T
    n_pairs = n_chunks // 2
    mesh = plsc.VectorSubcoreMesh(core_axis_name="c", subcore_axis_name="s")

    @functools.partial(
        pl.kernel,
        out_type=[
            jax.ShapeDtypeStruct((M, C), jnp.float32),
            jax.ShapeDtypeStruct((M, C), jnp.float32),
        ],
        mesh=mesh,
        scratch_types=[
            pltpu.VMEM((2, _T, C), jnp.float32),
            pltpu.VMEM((2, _T, C), jnp.float32),
            pltpu.VMEM((2, _H, C), jnp.float32),
            pltpu.VMEM((2, _H, C), jnp.float32),
            pltpu.VMEM((R,), jnp.float32),
            pltpu.VMEM((R,), jnp.float32),
            pltpu.VMEM((_L,), jnp.float32),
            pltpu.SemaphoreType.DMA,
            pltpu.SemaphoreType.DMA,
            pltpu.SemaphoreType.DMA,
            pltpu.SemaphoreType.DMA,
        ],
    )
    def sc_call(thr_hbm, m0_hbm, m1_hbm, x0_hbm, x1_hbm, o0_hbm, o1_hbm,
                x0c, x1c, o0c, o1c, m0all, m1all, thr_v,
                sem_in0, sem_in1, sem_out0, sem_out1):
        sem_in = (sem_in0, sem_in1)
        sem_out = (sem_out0, sem_out1)
        wid = lax.axis_index("s") * _NC + lax.axis_index("c")
        base_row = wid * R
        pltpu.sync_copy(thr_hbm, thr_v)
        pltpu.sync_copy(m0_hbm.at[pl.ds(base_row, R)], m0all)
        pltpu.sync_copy(m1_hbm.at[pl.ds(base_row, R)], m1all)
        thrv = thr_v[...]

        def start_in(c, i):
            row = base_row + c * _T
            s = sem_in[i]
            pltpu.make_async_copy(
                x0_hbm.at[pl.ds(row, _T)], x0c.at[i], s).start()
            pltpu.make_async_copy(
                x1_hbm.at[pl.ds(row, _T)], x1c.at[i], s).start()

        def wait_in(i):
            s = sem_in[i]
            pltpu.make_async_copy(
                x0_hbm.at[pl.ds(0, _T)], x0c.at[i], s).wait()
            pltpu.make_async_copy(
                x1_hbm.at[pl.ds(0, _T)], x1c.at[i], s).wait()

        def start_out(c, j):
            row = base_row + c * _T + j * _H
            s = sem_out[j]
            pltpu.make_async_copy(
                o0c.at[j], o0_hbm.at[pl.ds(row, _H)], s).start()
            pltpu.make_async_copy(
                o1c.at[j], o1_hbm.at[pl.ds(row, _H)], s).start()

        def wait_out(j):
            s = sem_out[j]
            pltpu.make_async_copy(
                o0c.at[j], o0_hbm.at[pl.ds(0, _H)], s).wait()
            pltpu.make_async_copy(
                o1c.at[j], o1_hbm.at[pl.ds(0, _H)], s).wait()

        def compute_half(mv0, mv1, i, j):
            xa = x0c.at[i]
            xb = x1c.at[i]
            oa = o0c.at[j]
            ob = o1c.at[j]
            for t in range(_H):
                lane = j * _H + t
                kv0 = jnp.broadcast_to(mv0[lane], (_L,)) >= thrv
                kv1 = jnp.broadcast_to(mv1[lane], (_L,)) >= thrv
                xat = xa.at[lane]
                xbt = xb.at[lane]
                oat = oa.at[t]
                obt = ob.at[t]

                @plsc.parallel_loop(0, C, step=_L, unroll=8)
                def jbody(off, kv0=kv0, kv1=kv1, xat=xat, xbt=xbt,
                          oat=oat, obt=obt):
                    sl = pl.ds(off, _L)
                    a = xat[sl]
                    b = xbt[sl]
                    oat[sl] = jnp.where(kv0, a, b)
                    obt[sl] = jnp.where(kv1, b, a)

        start_in(0, 0)

        def pair_body(k, carry):
            for i in range(2):
                c = 2 * k + i
                if i == 0:
                    start_in(c + 1, 1)
                else:
                    @pl.when(k < n_pairs - 1)
                    def _():
                        start_in(2 * k + 2, 0)
                wait_in(i)
                mv0 = m0all[pl.ds(c * _T, _L)]
                mv1 = m1all[pl.ds(c * _T, _L)]
                for j in range(2):
                    if i == 0:
                        @pl.when(k > 0)
                        def _(j=j):
                            wait_out(j)
                    else:
                        wait_out(j)
                    compute_half(mv0, mv1, i, j)
                    start_out(c, j)
            return carry

        lax.fori_loop(0, n_pairs, pair_body, 0)
        wait_out(0)
        wait_out(1)

    return sc_call


def kernel(x0, x1, mask0, mask1, mask_threshold):
    B, N, C = x0.shape
    M = B * N
    x0f = x0.reshape(M, C)
    x1f = x1.reshape(M, C)
    m0 = mask0.reshape(M)
    m1 = mask1.reshape(M)
    thr = jnp.full((_L,), mask_threshold, jnp.float32)
    o0, o1 = _make_sc_call(M, C)(thr, m0, m1, x0f, x1f)
    return o0.reshape(B, N, C), o1.reshape(B, N, C)
